# baseline (speedup 1.0000x reference)
import functools

import jax
import jax.numpy as jnp
from jax import lax
from jax.experimental import pallas as pl
from jax.experimental.pallas import tpu as pltpu
from jax.experimental.pallas import tpu_sc as plsc

_R = 65536
_D = 128
_NS = 64
_OUT = _D + _NS
_EPS = 1e-5
_NC = 2
_NSUB = 16
_NW = _NC * _NSUB
_RPW = _R // _NW
_G = 16
_NGRP = _RPW // _G
_HS = 66
_BW = 8
_NB = _D // _BW


def _sc_body(w_hbm, s_hbm, out_hbm,
             w0, s0, o0, w1, s1, o1, buf_v, hist_v, bsum_v, chk_v,
             sin0, sin1, sout0, sout1):
    wid = lax.axis_index("s") * _NC + lax.axis_index("c")
    lane = lax.iota(jnp.int32, 16)
    b128 = lane * _D
    b129 = lane * (_D + 1)
    b192 = lane * _OUT
    b66 = lane * _HS
    zeros_i = jnp.zeros((16,), jnp.int32)
    zeros_f = jnp.zeros((16,), jnp.float32)
    ones_i = jnp.ones((16,), jnp.int32)

    def in_slice(g):
        return pl.ds((wid * _RPW + g * _G) * _D, _G * _D)

    def out_slice(g):
        return pl.ds((wid * _RPW + g * _G) * _OUT, _G * _OUT)

    def start_in(g, wb, sb, sem):
        pltpu.async_copy(w_hbm.at[in_slice(g)], wb, sem)
        pltpu.async_copy(s_hbm.at[in_slice(g)], sb, sem)

    def wait_in(g, wb, sb, sem):
        pltpu.make_async_copy(w_hbm.at[in_slice(g)], wb, sem).wait()
        pltpu.make_async_copy(s_hbm.at[in_slice(g)], sb, sem).wait()

    def zero_hist():
        @plsc.parallel_loop(0, _NS)
        def _zh(m):
            plsc.store_scatter(hist_v, [b66 + m], zeros_i)

    zero_hist()

    def compute(wb, sb, ob):
        plsc.store_scatter(buf_v, [b129], zeros_f)

        @plsc.parallel_loop(0, _NB)
        def _pa(i):
            base = b128 + i * _BW
            w = [plsc.load_gather(wb, [base + t]) for t in range(_BW)]
            bs = ((w[0] + w[1]) + (w[2] + w[3])) + ((w[4] + w[5]) + (w[6] + w[7]))
            plsc.store_scatter(bsum_v, [lane + i * 16], bs)

        @plsc.parallel_loop(0, _NB, carry=zeros_f)
        def s_total(i, acc):
            plsc.store_scatter(chk_v, [lane + i * 16], acc)
            return acc + plsc.load_gather(bsum_v, [lane + i * 16])

        plsc.store_scatter(chk_v, [lane + _NB * 16], s_total)
        padv = jnp.maximum(_EPS - s_total, 0.0) * (1.0 / _D)
        inv = 1.0 / (s_total + padv * _D)

        def pcalc(craw, kp1f):
            cdf = (craw + kp1f * padv) * inv
            x = cdf * 64.0 - 0.5
            xi = x.astype(jnp.int32)
            return jnp.where(x > xi.astype(jnp.float32), xi + 1, xi), cdf

        @plsc.parallel_loop(0, _NB)
        def _pb(i):
            k0 = i * _BW
            k0f = k0.astype(jnp.float32)
            ck = plsc.load_gather(chk_v, [lane + i * 16])
            ck1 = plsc.load_gather(chk_v, [lane + (i + 1) * 16])
            pb_lo, _ = pcalc(ck, k0f)
            pb_hi, cdf_hi = pcalc(ck1, k0f + float(_BW))
            base = b128 + k0
            w = [plsc.load_gather(wb, [base + t]) for t in range(_BW - 1)]
            sv = [plsc.load_gather(sb, [base + t]) for t in range(_BW)]
            prev = pb_lo
            lp = zeros_f
            for t in range(_BW - 1):
                lp = lp + w[t]
                p_raw, cdf_t = pcalc(ck + lp, k0f + float(t + 1))
                p_t = jnp.minimum(jnp.maximum(p_raw, prev), pb_hi)
                plsc.store_scatter(buf_v, [b129 + (k0 + t + 1)], cdf_t)
                plsc.store_scatter(ob, [b192 + (k0 + t) + prev], sv[t])
                plsc.addupdate_scatter(hist_v, [b66 + p_t], ones_i)
                prev = p_t
            plsc.store_scatter(buf_v, [b129 + (k0 + _BW)], cdf_hi)
            plsc.store_scatter(ob, [b192 + (k0 + _BW - 1) + prev], sv[_BW - 1])
            plsc.addupdate_scatter(hist_v, [b66 + pb_hi], ones_i)

        @plsc.parallel_loop(0, _NS, unroll=4, carry=zeros_i)
        def _pc(j, accn):
            h = plsc.load_gather(hist_v, [b66 + j])
            n = accn + h
            nr = jnp.minimum(n + 1, _D - 1)
            cl = plsc.load_gather(buf_v, [b129 + n])
            cr = plsc.load_gather(buf_v, [b129 + nr])
            ol = plsc.load_gather(sb, [b128 + n])
            orr = plsc.load_gather(sb, [b128 + nr])
            u = (j.astype(jnp.float32) * 2.0 + 1.0) * (1.0 / 128.0)
            dd = cr - cl
            t = jnp.where(dd > 0, (u - cl) / dd, 0.0)
            t = jnp.clip(t, 0.0, 1.0)
            nv = ol + t * (orr - ol)
            plsc.store_scatter(ob, [b192 + (j + 1) + n], nv)
            return n

        zero_hist()

    start_in(0, w0, s0, sin0)
    start_in(1, w1, s1, sin1)
    nh = _NGRP // 2

    def _ring(h, c):
        g0 = h * 2
        g1 = g0 + 1
        wait_in(g0, w0, s0, sin0)

        @pl.when(h > 0)
        def _():
            pltpu.make_async_copy(o0, out_hbm.at[out_slice(g0 - 2)], sout0).wait()

        compute(w0, s0, o0)
        pltpu.async_copy(o0, out_hbm.at[out_slice(g0)], sout0)

        @pl.when(h < nh - 1)
        def _():
            start_in(g0 + 2, w0, s0, sin0)

        wait_in(g1, w1, s1, sin1)

        @pl.when(h > 0)
        def _():
            pltpu.make_async_copy(o1, out_hbm.at[out_slice(g1 - 2)], sout1).wait()

        compute(w1, s1, o1)
        pltpu.async_copy(o1, out_hbm.at[out_slice(g1)], sout1)

        @pl.when(h < nh - 1)
        def _():
            start_in(g1 + 2, w1, s1, sin1)

        return c

    lax.fori_loop(0, nh, _ring, 0)
    pltpu.make_async_copy(o0, out_hbm.at[out_slice(_NGRP - 2)], sout0).wait()
    pltpu.make_async_copy(o1, out_hbm.at[out_slice(_NGRP - 1)], sout1).wait()


_sc_kernel = functools.partial(
    pl.kernel,
    out_type=jax.ShapeDtypeStruct((_R * _OUT,), jnp.float32),
    mesh=plsc.VectorSubcoreMesh(
        core_axis_name="c", subcore_axis_name="s",
        num_cores=_NC, num_subcores=_NSUB),
    compiler_params=pltpu.CompilerParams(needs_layout_passes=False),
    scratch_types=[
        pltpu.VMEM((_G * _D,), jnp.float32),
        pltpu.VMEM((_G * _D,), jnp.float32),
        pltpu.VMEM((_G * _OUT,), jnp.float32),
        pltpu.VMEM((_G * _D,), jnp.float32),
        pltpu.VMEM((_G * _D,), jnp.float32),
        pltpu.VMEM((_G * _OUT,), jnp.float32),
        pltpu.VMEM((_G * (_D + 1),), jnp.float32),
        pltpu.VMEM((_G * _HS,), jnp.int32),
        pltpu.VMEM((_NB * 16,), jnp.float32),
        pltpu.VMEM(((_NB + 1) * 16,), jnp.float32),
        pltpu.SemaphoreType.DMA,
        pltpu.SemaphoreType.DMA,
        pltpu.SemaphoreType.DMA,
        pltpu.SemaphoreType.DMA,
    ],
)(_sc_body)


def kernel(weights, s_offsets):
    out = _sc_kernel(weights.reshape(-1), s_offsets.reshape(-1))
    return out.reshape(_R, _OUT)


# speedup vs baseline: 1.1157x; 1.1157x over previous
import functools

import jax
import jax.numpy as jnp
from jax import lax
from jax.experimental import pallas as pl
from jax.experimental.pallas import tpu as pltpu
from jax.experimental.pallas import tpu_sc as plsc

_R = 65536
_D = 128
_NS = 64
_OUT = _D + _NS
_EPS = 1e-5
_NC = 2
_NSUB = 16
_NW = _NC * _NSUB
_RPW = _R // _NW
_G = 16
_NGRP = _RPW // _G
_HS = 66


def _sc_body(w_hbm, s_hbm, out_hbm,
             w0, s0, o0, w1, s1, o1, buf_v, hist_v,
             sin0, sin1, sout0, sout1):
    wid = lax.axis_index("s") * _NC + lax.axis_index("c")
    lane = lax.iota(jnp.int32, 16)
    b128 = lane * _D
    b129 = lane * (_D + 1)
    b192 = lane * _OUT
    b66 = lane * _HS
    zeros_i = jnp.zeros((16,), jnp.int32)
    zeros_f = jnp.zeros((16,), jnp.float32)
    ones_i = jnp.ones((16,), jnp.int32)

    def in_slice(g):
        return pl.ds((wid * _RPW + g * _G) * _D, _G * _D)

    def out_slice(g):
        return pl.ds((wid * _RPW + g * _G) * _OUT, _G * _OUT)

    def start_in(g, wb, sb, sem):
        pltpu.async_copy(w_hbm.at[in_slice(g)], wb, sem)
        pltpu.async_copy(s_hbm.at[in_slice(g)], sb, sem)

    def wait_in(g, wb, sb, sem):
        pltpu.make_async_copy(w_hbm.at[in_slice(g)], wb, sem).wait()
        pltpu.make_async_copy(s_hbm.at[in_slice(g)], sb, sem).wait()

    def zero_hist():
        @plsc.parallel_loop(0, _NS)
        def _zh(m):
            plsc.store_scatter(hist_v, [b66 + m], zeros_i)

    zero_hist()

    def compute(wb, sb, ob):
        plsc.store_scatter(buf_v, [b129], zeros_f)

        @plsc.parallel_loop(0, _D, unroll=8, carry=zeros_f)
        def s_total(k, acc):
            return acc + plsc.load_gather(wb, [b128 + k])

        padv = jnp.maximum(_EPS - s_total, 0.0) * (1.0 / _D)
        inv = 1.0 / (s_total + padv * _D)

        @plsc.parallel_loop(0, _D, unroll=8, carry=(zeros_f, zeros_i))
        def _pb(k, carry):
            acc, prev_p = carry
            acc = acc + plsc.load_gather(wb, [b128 + k])
            kp1 = (k + 1).astype(jnp.float32)
            cdf = (acc + kp1 * padv) * inv
            plsc.store_scatter(buf_v, [b129 + (k + 1)], cdf)
            p = (cdf * 64.0 + 0.5).astype(jnp.int32)
            sk = plsc.load_gather(sb, [b128 + k])
            plsc.store_scatter(ob, [b192 + k + prev_p], sk)
            plsc.addupdate_scatter(hist_v, [b66 + p], ones_i)
            return acc, p

        @plsc.parallel_loop(0, _NS, unroll=8, carry=zeros_i)
        def _pc(j, accn):
            h = plsc.load_gather(hist_v, [b66 + j])
            plsc.store_scatter(hist_v, [b66 + j], zeros_i)
            n = accn + h
            nr = jnp.minimum(n + 1, _D - 1)
            cl = plsc.load_gather(buf_v, [b129 + n])
            cr = plsc.load_gather(buf_v, [b129 + nr])
            ol = plsc.load_gather(sb, [b128 + n])
            orr = plsc.load_gather(sb, [b128 + nr])
            u = (j.astype(jnp.float32) * 2.0 + 1.0) * (1.0 / 128.0)
            dd = cr - cl
            t = jnp.where(dd > 0, (u - cl) / dd, 0.0)
            t = jnp.clip(t, 0.0, 1.0)
            nv = ol + t * (orr - ol)
            plsc.store_scatter(ob, [b192 + (j + 1) + n], nv)
            return n

    start_in(0, w0, s0, sin0)
    start_in(1, w1, s1, sin1)
    nh = _NGRP // 2

    def _ring(h, c):
        g0 = h * 2
        g1 = g0 + 1
        wait_in(g0, w0, s0, sin0)

        @pl.when(h > 0)
        def _():
            pltpu.make_async_copy(o0, out_hbm.at[out_slice(g0 - 2)], sout0).wait()

        compute(w0, s0, o0)
        pltpu.async_copy(o0, out_hbm.at[out_slice(g0)], sout0)

        @pl.when(h < nh - 1)
        def _():
            start_in(g0 + 2, w0, s0, sin0)

        wait_in(g1, w1, s1, sin1)

        @pl.when(h > 0)
        def _():
            pltpu.make_async_copy(o1, out_hbm.at[out_slice(g1 - 2)], sout1).wait()

        compute(w1, s1, o1)
        pltpu.async_copy(o1, out_hbm.at[out_slice(g1)], sout1)

        @pl.when(h < nh - 1)
        def _():
            start_in(g1 + 2, w1, s1, sin1)

        return c

    lax.fori_loop(0, nh, _ring, 0)
    pltpu.make_async_copy(o0, out_hbm.at[out_slice(_NGRP - 2)], sout0).wait()
    pltpu.make_async_copy(o1, out_hbm.at[out_slice(_NGRP - 1)], sout1).wait()


_sc_kernel = functools.partial(
    pl.kernel,
    out_type=jax.ShapeDtypeStruct((_R * _OUT,), jnp.float32),
    mesh=plsc.VectorSubcoreMesh(
        core_axis_name="c", subcore_axis_name="s",
        num_cores=_NC, num_subcores=_NSUB),
    compiler_params=pltpu.CompilerParams(needs_layout_passes=False),
    scratch_types=[
        pltpu.VMEM((_G * _D,), jnp.float32),
        pltpu.VMEM((_G * _D,), jnp.float32),
        pltpu.VMEM((_G * _OUT,), jnp.float32),
        pltpu.VMEM((_G * _D,), jnp.float32),
        pltpu.VMEM((_G * _D,), jnp.float32),
        pltpu.VMEM((_G * _OUT,), jnp.float32),
        pltpu.VMEM((_G * (_D + 1),), jnp.float32),
        pltpu.VMEM((_G * _HS,), jnp.int32),
        pltpu.SemaphoreType.DMA,
        pltpu.SemaphoreType.DMA,
        pltpu.SemaphoreType.DMA,
        pltpu.SemaphoreType.DMA,
    ],
)(_sc_body)


def kernel(weights, s_offsets):
    out = _sc_kernel(weights.reshape(-1), s_offsets.reshape(-1))
    return out.reshape(_R, _OUT)
